# Initial kernel scaffold; baseline (speedup 1.0000x reference)
import jax
import jax.numpy as jnp
from jax.experimental import pallas as pl


def kernel(feature_map, gt_boxes, gt_classes, W1, b1, Ws, bs, Wc, bc, anchors):
    raise NotImplementedError("write your pallas kernel here")


# speedup vs baseline: 7.0944x; 7.0944x over previous
import functools

import jax
import jax.numpy as jnp
import numpy as np
from jax.experimental import pallas as pl
from jax.experimental.pallas import tpu as pltpu

IMG_SIZE = 512.0
NMS_THRESH = 0.7
PRE_N = 1000
POST_N = 300
NPOS = 1024
NA = 9
NBOX = NPOS * NA
NT = NBOX // 128


def _heads_body(hid_ref, w_ref, b_ref, anc_ref, fg_ref, box_ref):
    h = hid_ref[0].astype(jnp.bfloat16)
    w = w_ref[:].astype(jnp.bfloat16)
    head = jnp.dot(h, w, preferred_element_type=jnp.float32) + b_ref[:]
    s0 = head[:, 0:128]
    s1 = head[:, 128:256]
    m = jnp.maximum(s0, s1)
    e0 = jnp.exp(s0 - m)
    e1 = jnp.exp(s1 - m)
    fg_ref[0] = e0 / (e0 + e1)
    dx = head[:, 256:384]
    dy = head[:, 384:512]
    dw = head[:, 512:640]
    dh = head[:, 640:768]
    aw = anc_ref[0]
    ah = anc_ref[1]
    acx = anc_ref[2]
    acy = anc_ref[3]
    pcx = dx * aw + acx
    pcy = dy * ah + acy
    pw = jnp.exp(jnp.clip(dw, -10.0, 4.0)) * aw
    ph = jnp.exp(jnp.clip(dh, -10.0, 4.0)) * ah
    box_ref[0, 0] = jnp.clip(pcx - 0.5 * pw, 0.0, IMG_SIZE)
    box_ref[0, 1] = jnp.clip(pcy - 0.5 * ph, 0.0, IMG_SIZE)
    box_ref[0, 2] = jnp.clip(pcx + 0.5 * pw, 0.0, IMG_SIZE)
    box_ref[0, 3] = jnp.clip(pcy + 0.5 * ph, 0.0, IMG_SIZE)


def _select_body(fgt_ref, fgr_ref, v_ref, out_ref, rank_ref, m_ref):
    f32 = jnp.float32
    fg_col = fgt_ref[0]
    jidx = jax.lax.broadcasted_iota(jnp.int32, (NBOX, 1), 0).astype(f32)

    def rank_blk(ib, _):
        fi = fgr_ref[0, ib]
        iidx = (ib * 128).astype(f32) + jax.lax.broadcasted_iota(
            jnp.int32, (1, 128), 1).astype(f32)
        gt = (fg_col > fi).astype(f32)
        tie = jnp.logical_and(fg_col == fi, jidx < iidx).astype(f32)
        rank_ref[ib] = jnp.sum(gt + tie, axis=0, keepdims=True)
        return 0

    jax.lax.fori_loop(0, NT, rank_blk, 0)

    diota = jax.lax.broadcasted_iota(jnp.int32, (NPOS, 1), 0).astype(f32)

    def gather_blk(jt, top):
        rrow = rank_ref[jt]
        oh = jnp.logical_and(diota == rrow, rrow < float(PRE_N)).astype(f32)
        return top + jnp.dot(oh, v_ref[0, jt], preferred_element_type=f32)

    top = jax.lax.fori_loop(
        0, NT, gather_blk, jnp.zeros((NPOS, 8), f32))
    topT = top.T

    jio = jax.lax.broadcasted_iota(jnp.int32, (1, NPOS), 1).astype(f32)
    for rb in range(4):
        i0 = rb * 256
        tb = top[i0:i0 + 256]
        x1c, y1c = tb[:, 0:1], tb[:, 1:2]
        x2c, y2c = tb[:, 2:3], tb[:, 3:4]
        arc = (x2c - x1c) * (y2c - y1c)
        x1r, y1r = topT[0:1], topT[1:2]
        x2r, y2r = topT[2:3], topT[3:4]
        arr = (x2r - x1r) * (y2r - y1r)
        xx1 = jnp.maximum(x1c, x1r)
        yy1 = jnp.maximum(y1c, y1r)
        xx2 = jnp.minimum(x2c, x2r)
        yy2 = jnp.minimum(y2c, y2r)
        inter = jnp.maximum(xx2 - xx1, 0.0) * jnp.maximum(yy2 - yy1, 0.0)
        iou = inter / (arc + arr - inter + 1e-9)
        iio = float(i0) + jax.lax.broadcasted_iota(jnp.int32, (256, 1), 0).astype(f32)
        m_ref[i0:i0 + 256, :] = jnp.logical_and(
            iou > NMS_THRESH, iio < jio).astype(f32)

    mv = m_ref[:]

    def cond(c):
        _, changed, it = c
        return jnp.logical_and(changed, it < NPOS)

    def body(c):
        k, _, it = c
        counts = jnp.dot(k, mv, preferred_element_type=f32)
        knew = (counts == 0.0).astype(f32)
        return knew, jnp.sum(jnp.abs(knew - k)) > 0.0, it + 1

    keep, _, _ = jax.lax.while_loop(
        cond, body, (jnp.ones((1, NPOS), f32), jnp.bool_(True),
                     jnp.int32(0)))

    valid = (jio < float(PRE_N)).astype(f32)
    kv = keep * valid
    nk = (1.0 - keep) * valid
    iioc = jax.lax.broadcasted_iota(jnp.int32, (NPOS, 1), 0).astype(f32)
    tri = (iioc <= jio).astype(f32)
    ck = jnp.dot(kv, tri, preferred_element_type=f32)
    cn = jnp.dot(nk, tri, preferred_element_type=f32)
    K = jnp.sum(kv)
    dest = jnp.where(kv > 0.0, ck - 1.0,
                     jnp.where(nk > 0.0, K + cn - 1.0, 2048.0))
    d512 = jax.lax.broadcasted_iota(jnp.int32, (512, 1), 0).astype(f32)
    oh = (d512 == dest).astype(f32)
    res = jnp.dot(oh, top, preferred_element_type=f32)
    out_ref[0] = res[0:POST_N, 0:4]


@functools.partial(jax.jit, static_argnums=())
def kernel(feature_map, gt_boxes, gt_classes, W1, b1, Ws, bs, Wc, bc,
           anchors):
    del gt_boxes, gt_classes
    n = feature_map.shape[0]
    f32 = jnp.float32

    y = jax.lax.conv_general_dilated(
        feature_map, W1, window_strides=(1, 1),
        padding=((1, 1), (1, 1)),
        dimension_numbers=('NCHW', 'OIHW', 'NCHW'))
    hid = jax.nn.relu(y + b1[None, :, None, None])
    hid3 = jnp.transpose(hid, (0, 2, 3, 1)).reshape(n, NPOS, 512)

    wh = jnp.zeros((512, 768), f32)
    ws2 = Ws[:, :, 0, 0]
    wc2 = Wc[:, :, 0, 0]
    wh = wh.at[:, 0:9].set(ws2[0::2].T)
    wh = wh.at[:, 128:137].set(ws2[1::2].T)
    for c in range(4):
        wh = wh.at[:, 256 + 128 * c:256 + 128 * c + 9].set(wc2[c::4].T)
    bh = jnp.zeros((1, 768), f32)
    bh = bh.at[0, 0:9].set(bs[0::2])
    bh = bh.at[0, 128:137].set(bs[1::2])
    for c in range(4):
        bh = bh.at[0, 256 + 128 * c:256 + 128 * c + 9].set(bc[c::4])

    a9 = anchors.reshape(NPOS, NA, 4)
    aw = a9[:, :, 2] - a9[:, :, 0]
    ah = a9[:, :, 3] - a9[:, :, 1]
    acx = a9[:, :, 0] + 0.5 * aw
    acy = a9[:, :, 1] + 0.5 * ah
    anc = jnp.zeros((4, NPOS, 128), f32)
    anc = anc.at[0, :, 0:9].set(aw)
    anc = anc.at[1, :, 0:9].set(ah)
    anc = anc.at[2, :, 0:9].set(acx)
    anc = anc.at[3, :, 0:9].set(acy)

    fg, boxf = pl.pallas_call(
        _heads_body,
        grid=(n,),
        in_specs=[
            pl.BlockSpec((1, NPOS, 512), lambda i: (i, 0, 0)),
            pl.BlockSpec((512, 768), lambda i: (0, 0)),
            pl.BlockSpec((1, 768), lambda i: (0, 0)),
            pl.BlockSpec((4, NPOS, 128), lambda i: (0, 0, 0)),
        ],
        out_specs=[
            pl.BlockSpec((1, NPOS, 128), lambda i: (i, 0, 0)),
            pl.BlockSpec((1, 4, NPOS, 128), lambda i: (i, 0, 0, 0)),
        ],
        out_shape=[
            jax.ShapeDtypeStruct((n, NPOS, 128), f32),
            jax.ShapeDtypeStruct((n, 4, NPOS, 128), f32),
        ],
    )(hid3, wh, bh, anc)

    fgflat = fg[:, :, 0:9].reshape(n, NBOX)
    bflat = boxf[:, :, :, 0:9]
    v = jnp.zeros((n, NBOX, 8), f32)
    v = v.at[:, :, 0:4].set(
        jnp.transpose(bflat, (0, 2, 3, 1)).reshape(n, NBOX, 4))

    fgt = fgflat.reshape(n, NBOX, 1)
    fgr = fgflat.reshape(n, NT, 1, 128)
    v3 = v.reshape(n, NT, 128, 8)

    rois = pl.pallas_call(
        _select_body,
        grid=(n,),
        in_specs=[
            pl.BlockSpec((1, NBOX, 1), lambda i: (i, 0, 0)),
            pl.BlockSpec((1, NT, 1, 128), lambda i: (i, 0, 0, 0)),
            pl.BlockSpec((1, NT, 128, 8), lambda i: (i, 0, 0, 0)),
        ],
        out_specs=pl.BlockSpec((1, POST_N, 4), lambda i: (i, 0, 0)),
        out_shape=jax.ShapeDtypeStruct((n, POST_N, 4), f32),
        scratch_shapes=[
            pltpu.VMEM((NT, 1, 128), f32),
            pltpu.VMEM((NPOS, NPOS), f32),
        ],
    )(fgt, fgr, v3)
    return rois


# speedup vs baseline: 13.3233x; 1.8780x over previous
import functools

import jax
import jax.numpy as jnp
import numpy as np
from jax.experimental import pallas as pl
from jax.experimental.pallas import tpu as pltpu

IMG_SIZE = 512.0
NMS_THRESH = 0.7
PRE_N = 1000
POST_N = 300
NPOS = 1024
NA = 9
NBOX = NPOS * NA
NT = NBOX // 128


def _select_body(bs_ref, out_ref, m_ref):
    f32 = jnp.float32
    top = bs_ref[0]
    topT = top.T

    jio = jax.lax.broadcasted_iota(jnp.int32, (1, NPOS), 1).astype(f32)
    for rb in range(4):
        i0 = rb * 256
        tb = top[i0:i0 + 256]
        x1c, y1c = tb[:, 0:1], tb[:, 1:2]
        x2c, y2c = tb[:, 2:3], tb[:, 3:4]
        arc = (x2c - x1c) * (y2c - y1c)
        x1r, y1r = topT[0:1], topT[1:2]
        x2r, y2r = topT[2:3], topT[3:4]
        arr = (x2r - x1r) * (y2r - y1r)
        xx1 = jnp.maximum(x1c, x1r)
        yy1 = jnp.maximum(y1c, y1r)
        xx2 = jnp.minimum(x2c, x2r)
        yy2 = jnp.minimum(y2c, y2r)
        inter = jnp.maximum(xx2 - xx1, 0.0) * jnp.maximum(yy2 - yy1, 0.0)
        iou = inter / (arc + arr - inter + 1e-9)
        iio = float(i0) + jax.lax.broadcasted_iota(jnp.int32, (256, 1), 0).astype(f32)
        m_ref[i0:i0 + 256, :] = jnp.logical_and(
            iou > NMS_THRESH, iio < jio).astype(f32)

    mv = m_ref[:]

    def cond(c):
        _, changed, it = c
        return jnp.logical_and(changed, it < NPOS)

    def body(c):
        k, _, it = c
        counts = jnp.dot(k, mv, preferred_element_type=f32)
        knew = (counts == 0.0).astype(f32)
        return knew, jnp.sum(jnp.abs(knew - k)) > 0.0, it + 1

    keep, _, _ = jax.lax.while_loop(
        cond, body, (jnp.ones((1, NPOS), f32), jnp.bool_(True),
                     jnp.int32(0)))

    valid = (jio < float(PRE_N)).astype(f32)
    kv = keep * valid
    nk = (1.0 - keep) * valid
    iioc = jax.lax.broadcasted_iota(jnp.int32, (NPOS, 1), 0).astype(f32)
    tri = (iioc <= jio).astype(f32)
    ck = jnp.dot(kv, tri, preferred_element_type=f32)
    cn = jnp.dot(nk, tri, preferred_element_type=f32)
    K = jnp.sum(kv)
    dest = jnp.where(kv > 0.0, ck - 1.0,
                     jnp.where(nk > 0.0, K + cn - 1.0, 2048.0))
    d512 = jax.lax.broadcasted_iota(jnp.int32, (512, 1), 0).astype(f32)
    oh = (d512 == dest).astype(f32)
    res = jnp.dot(oh, top, preferred_element_type=f32,
                  precision=jax.lax.Precision.HIGHEST)
    out_ref[0] = res[0:POST_N, 0:4]


@functools.partial(jax.jit, static_argnums=())
def kernel(feature_map, gt_boxes, gt_classes, W1, b1, Ws, bs, Wc, bc,
           anchors):
    del gt_boxes, gt_classes
    n = feature_map.shape[0]
    f32 = jnp.float32

    def conv2d(x, w, b, pad):
        y = jax.lax.conv_general_dilated(
            x, w, window_strides=(1, 1),
            padding=((pad, pad), (pad, pad)),
            dimension_numbers=('NCHW', 'OIHW', 'NCHW'))
        return y + b[None, :, None, None]

    out = jax.nn.relu(conv2d(feature_map, W1, b1, 1))
    score_map = conv2d(out, Ws, bs, 0)
    coeff_map = conv2d(out, Wc, bc, 0)
    bbox_score = jnp.transpose(score_map, (0, 2, 3, 1)).reshape(n, -1, 2)
    bbox_coeff = jnp.transpose(coeff_map, (0, 2, 3, 1)).reshape(n, -1, 4)
    fg = jax.nn.softmax(bbox_score, axis=-1)[..., 0]
    aw = anchors[:, 2] - anchors[:, 0]
    ah = anchors[:, 3] - anchors[:, 1]
    acx = anchors[:, 0] + 0.5 * aw
    acy = anchors[:, 1] + 0.5 * ah
    dx, dy = bbox_coeff[..., 0], bbox_coeff[..., 1]
    dw, dh = bbox_coeff[..., 2], bbox_coeff[..., 3]
    pcx = dx * aw + acx
    pcy = dy * ah + acy
    pw = jnp.exp(jnp.clip(dw, -10.0, 4.0)) * aw
    ph = jnp.exp(jnp.clip(dh, -10.0, 4.0)) * ah
    boxes = jnp.stack([pcx - 0.5 * pw, pcy - 0.5 * ph,
                       pcx + 0.5 * pw, pcy + 0.5 * ph], axis=-1)
    boxes = jnp.clip(boxes, 0.0, IMG_SIZE)

    bsel = []
    for i in range(n):
        _, idx = jax.lax.top_k(fg[i], PRE_N)
        bsel.append(jnp.take(boxes[i], idx, axis=0))
    b = jnp.stack(bsel, axis=0)
    bpad = jnp.zeros((n, NPOS, 8), f32).at[:, 0:PRE_N, 0:4].set(b)

    rois = pl.pallas_call(
        _select_body,
        grid=(n,),
        in_specs=[pl.BlockSpec((1, NPOS, 8), lambda i: (i, 0, 0))],
        out_specs=pl.BlockSpec((1, POST_N, 4), lambda i: (i, 0, 0)),
        out_shape=jax.ShapeDtypeStruct((n, POST_N, 4), f32),
        scratch_shapes=[pltpu.VMEM((NPOS, NPOS), f32)],
    )(bpad)
    return rois
